# Initial kernel scaffold; baseline (speedup 1.0000x reference)
import jax
import jax.numpy as jnp
from jax.experimental import pallas as pl


def kernel(x, edge_index, W1, b1, Wmu, bmu, Wls, bls):
    raise NotImplementedError("write your pallas kernel here")


# speedup vs baseline: 9.6590x; 9.6590x over previous
import functools

import jax
import jax.numpy as jnp
from jax import lax
from jax.experimental import pallas as pl
from jax.experimental.pallas import tpu as pltpu
from jax.experimental.pallas import tpu_sc as plsc

N = 10000
E = 320000
D = 128
H2 = 32
C = 16
MAX_LOGSTD = 10.0
EPS = 1e-15

N8 = N + 8
NW = 32
CHUNK = 128
KC = 80
PT = KC * CHUNK
EP = NW * PT
GK = 8
KG = KC // GK

_mesh = plsc.VectorSubcoreMesh(core_axis_name="c", subcore_axis_name="s")
_sc_params = pltpu.CompilerParams(use_tc_tiling_on_sc=False)


def _wid():
    return lax.axis_index("s") * 2 + lax.axis_index("c")


@functools.partial(
    pl.kernel,
    out_type=jax.ShapeDtypeStruct((2, N8), jnp.float32),
    mesh=_mesh,
    compiler_params=_sc_params,
    scratch_types=[
        pltpu.VMEM((KC, CHUNK), jnp.int32),
        pltpu.VMEM((CHUNK,), jnp.float32),
        pltpu.VMEM_SHARED((N8,), jnp.float32),
    ],
)
def _sc_degree(dst_hbm, zeros1_hbm, ones_hbm, out_hbm, idx_v, ones_v, acc_sh):
    c = lax.axis_index("c")
    s = lax.axis_index("s")
    w = _wid()
    pltpu.sync_copy(dst_hbm.at[w], idx_v)
    pltpu.sync_copy(ones_hbm, ones_v)

    @pl.when(s == 0)
    def _():
        pltpu.sync_copy(zeros1_hbm, acc_sh)

    plsc.subcore_barrier()

    def body(j, carry):
        pltpu.sync_copy(ones_v, acc_sh.at[idx_v.at[j]], add=True)
        return carry

    lax.fori_loop(0, KC, body, 0)
    plsc.subcore_barrier()

    @pl.when(s == 0)
    def _():
        pltpu.sync_copy(acc_sh, out_hbm.at[c])


@functools.partial(
    pl.kernel,
    out_type=jax.ShapeDtypeStruct((2, N8, H2), jnp.float32),
    mesh=_mesh,
    compiler_params=_sc_params,
    scratch_types=[
        pltpu.VMEM((KC, CHUNK), jnp.int32),
        pltpu.VMEM((KC, CHUNK), jnp.int32),
        pltpu.VMEM((GK, CHUNK, H2), jnp.float32),
        pltpu.VMEM_SHARED((N8, H2), jnp.float32),
        pltpu.SemaphoreType.DMA,
    ],
)
def _sc_agg(y_hbm, src_hbm, dst_hbm, zeros2_hbm, out_hbm,
            src_v, dst_v, rows_v, acc_sh, sem):
    c = lax.axis_index("c")
    s = lax.axis_index("s")
    w = _wid()
    pltpu.sync_copy(src_hbm.at[w], src_v)
    pltpu.sync_copy(dst_hbm.at[w], dst_v)

    @pl.when(s == 0)
    def _():
        pltpu.sync_copy(zeros2_hbm, acc_sh)

    plsc.subcore_barrier()

    def group(g, carry):
        base = g * GK
        descs = []
        for b in range(GK):
            descs.append(
                pltpu.async_copy(y_hbm.at[src_v.at[base + b]], rows_v.at[b], sem))
        for d in descs:
            d.wait()
        for b in range(GK):
            pltpu.sync_copy(rows_v.at[b], acc_sh.at[dst_v.at[base + b]], add=True)
        return carry

    lax.fori_loop(0, KG, group, 0)
    plsc.subcore_barrier()

    @pl.when(s == 0)
    def _():
        pltpu.sync_copy(acc_sh, out_hbm.at[c])


@functools.partial(
    pl.kernel,
    out_type=jax.ShapeDtypeStruct((4, NW, KG, GK, CHUNK, C), jnp.float32),
    mesh=_mesh,
    compiler_params=_sc_params,
    scratch_types=[
        pltpu.VMEM((KC, CHUNK), jnp.int32),
        pltpu.VMEM((GK, CHUNK, C), jnp.float32),
        pltpu.SemaphoreType.DMA,
    ],
)
def _sc_decode_gather(z_hbm, idx_hbm, out_hbm, idx_v, rows_v, sem):
    w = _wid()
    for t in range(4):
        pltpu.sync_copy(idx_hbm.at[t, w], idx_v)

        def group(g, carry, t=t):
            base = g * GK
            descs = []
            for b in range(GK):
                descs.append(
                    pltpu.async_copy(z_hbm.at[idx_v.at[base + b]], rows_v.at[b], sem))
            for d in descs:
                d.wait()
            pltpu.sync_copy(rows_v, out_hbm.at[t, w, g])
            return carry

        lax.fori_loop(0, KG, group, 0)


_BN = 1000
_GN = N // _BN


def _tc_b_body(degt_ref, x_ref, w1_ref, y1_ref):
    deg = degt_ref[:, 0:1] + degt_ref[:, 1:2] + 1.0
    dinv = lax.rsqrt(deg)
    xw = jnp.dot(x_ref[:], w1_ref[:], preferred_element_type=jnp.float32)
    y1_ref[:] = xw * dinv


def _tc_d_body(aggp_ref, degt_ref, y1_ref, b1_ref, y2_ref):
    deg = degt_ref[:, 0:1] + degt_ref[:, 1:2] + 1.0
    dinv = lax.rsqrt(deg)
    agg = aggp_ref[0] + aggp_ref[1] + y1_ref[:]
    h = jnp.maximum(agg * dinv + b1_ref[:], 0.0)
    y2_ref[:] = h * dinv


def _tc_f_body(aggp_ref, degt_ref, y2_ref, wmu_ref, bmu_ref, wls_ref, bls_ref,
               z_ref, kl_ref):
    i = pl.program_id(0)
    deg = degt_ref[:, 0:1] + degt_ref[:, 1:2] + 1.0
    dinv = lax.rsqrt(deg)
    hg = (aggp_ref[0] + aggp_ref[1] + y2_ref[:]) * dinv
    mu = jnp.dot(hg, wmu_ref[:], preferred_element_type=jnp.float32) + bmu_ref[:]
    ls = jnp.minimum(
        jnp.dot(hg, wls_ref[:], preferred_element_type=jnp.float32) + bls_ref[:],
        MAX_LOGSTD)
    z_ref[:] = mu
    klb = jnp.sum(1.0 + 2.0 * ls - mu * mu - jnp.exp(2.0 * ls))

    @pl.when(i == 0)
    def _():
        kl_ref[...] = jnp.zeros((1, 1), jnp.float32)

    kl_ref[...] += klb


_BE = 2000
_GE = E // _BE


def _tc_h_body(r_ref, kl_ref, loss_ref):
    i = pl.program_id(0)
    pos = jnp.sum(r_ref[0] * r_ref[1], axis=1, keepdims=True)
    neg = jnp.sum(r_ref[2] * r_ref[3], axis=1, keepdims=True)
    pls = jnp.sum(jnp.log(jax.nn.sigmoid(pos) + EPS))
    nls = jnp.sum(jnp.log(1.0 - jax.nn.sigmoid(neg) + EPS))

    @pl.when(i == 0)
    def _():
        loss_ref[...] = -0.5 * kl_ref[...] / (N * N)

    loss_ref[...] += -(pls + nls) / E


def _pad_idx(a):
    pad = jnp.full((EP - E,), N, jnp.int32)
    return jnp.concatenate([a.astype(jnp.int32), pad]).reshape(NW, KC, CHUNK)


def kernel(x, edge_index, W1, b1, Wmu, bmu, Wls, bls):
    src = edge_index[0]
    dst = edge_index[1]
    neg = jax.random.randint(jax.random.key(1), (2, E), 0, N, dtype=src.dtype)
    srcp = _pad_idx(src)
    dstp = _pad_idx(dst)
    dec_idx = jnp.stack([srcp, dstp, _pad_idx(neg[0]), _pad_idx(neg[1])])
    zeros1 = jnp.zeros((N8,), jnp.float32)
    zeros2 = jnp.zeros((N8, H2), jnp.float32)
    ones = jnp.ones((CHUNK,), jnp.float32)

    degp = _sc_degree(dstp, zeros1, ones)
    degt = degp[:, :N].T

    y1 = pl.pallas_call(
        _tc_b_body,
        grid=(_GN,),
        in_specs=[
            pl.BlockSpec((_BN, 2), lambda i: (i, 0)),
            pl.BlockSpec((_BN, D), lambda i: (i, 0)),
            pl.BlockSpec((D, H2), lambda i: (0, 0)),
        ],
        out_specs=pl.BlockSpec((_BN, H2), lambda i: (i, 0)),
        out_shape=jax.ShapeDtypeStruct((N, H2), jnp.float32),
    )(degt, x, W1)
    y1p = jnp.concatenate([y1, jnp.zeros((8, H2), jnp.float32)])

    agg1 = _sc_agg(y1p, srcp, dstp, zeros2)

    y2 = pl.pallas_call(
        _tc_d_body,
        grid=(_GN,),
        in_specs=[
            pl.BlockSpec((2, _BN, H2), lambda i: (0, i, 0)),
            pl.BlockSpec((_BN, 2), lambda i: (i, 0)),
            pl.BlockSpec((_BN, H2), lambda i: (i, 0)),
            pl.BlockSpec((1, H2), lambda i: (0, 0)),
        ],
        out_specs=pl.BlockSpec((_BN, H2), lambda i: (i, 0)),
        out_shape=jax.ShapeDtypeStruct((N, H2), jnp.float32),
    )(agg1, degt, y1, b1.reshape(1, H2))
    y2p = jnp.concatenate([y2, jnp.zeros((8, H2), jnp.float32)])

    agg2 = _sc_agg(y2p, srcp, dstp, zeros2)

    z, kl = pl.pallas_call(
        _tc_f_body,
        grid=(_GN,),
        in_specs=[
            pl.BlockSpec((2, _BN, H2), lambda i: (0, i, 0)),
            pl.BlockSpec((_BN, 2), lambda i: (i, 0)),
            pl.BlockSpec((_BN, H2), lambda i: (i, 0)),
            pl.BlockSpec((H2, C), lambda i: (0, 0)),
            pl.BlockSpec((1, C), lambda i: (0, 0)),
            pl.BlockSpec((H2, C), lambda i: (0, 0)),
            pl.BlockSpec((1, C), lambda i: (0, 0)),
        ],
        out_specs=[
            pl.BlockSpec((_BN, C), lambda i: (i, 0)),
            pl.BlockSpec((1, 1), lambda i: (0, 0)),
        ],
        out_shape=[
            jax.ShapeDtypeStruct((N, C), jnp.float32),
            jax.ShapeDtypeStruct((1, 1), jnp.float32),
        ],
    )(agg2, degt, y2, Wmu, bmu.reshape(1, C), Wls, bls.reshape(1, C))

    zp = jnp.concatenate([z, jnp.zeros((8, C), jnp.float32)])
    rows = _sc_decode_gather(zp, dec_idx)
    rows = rows.reshape(4, EP, C)

    loss = pl.pallas_call(
        _tc_h_body,
        grid=(_GE,),
        in_specs=[
            pl.BlockSpec((4, _BE, C), lambda i: (0, i, 0)),
            pl.BlockSpec((1, 1), lambda i: (0, 0)),
        ],
        out_specs=pl.BlockSpec((1, 1), lambda i: (0, 0)),
        out_shape=jax.ShapeDtypeStruct((1, 1), jnp.float32),
    )(rows, kl)

    return (z, loss[0, 0])


# speedup vs baseline: 10.0131x; 1.0367x over previous
import functools

import jax
import jax.numpy as jnp
from jax import lax
from jax.experimental import pallas as pl
from jax.experimental.pallas import tpu as pltpu
from jax.experimental.pallas import tpu_sc as plsc

N = 10000
E = 320000
D = 128
H2 = 32
C = 16
MAX_LOGSTD = 10.0
EPS = 1e-15

N8 = N + 8
NW = 32
CHUNK = 128
KC = 80
PT = KC * CHUNK
EP = NW * PT
GK = 8
KG = KC // GK

_mesh = plsc.VectorSubcoreMesh(core_axis_name="c", subcore_axis_name="s")
_sc_params = pltpu.CompilerParams(use_tc_tiling_on_sc=False)


def _wid():
    return lax.axis_index("s") * 2 + lax.axis_index("c")


@functools.partial(
    pl.kernel,
    out_type=jax.ShapeDtypeStruct((2, N8), jnp.float32),
    mesh=_mesh,
    compiler_params=_sc_params,
    scratch_types=[
        pltpu.VMEM((KC, CHUNK), jnp.int32),
        pltpu.VMEM((CHUNK,), jnp.float32),
        pltpu.VMEM_SHARED((N8,), jnp.float32),
    ],
)
def _sc_degree(dst_hbm, zeros1_hbm, ones_hbm, out_hbm, idx_v, ones_v, acc_sh):
    c = lax.axis_index("c")
    s = lax.axis_index("s")
    w = _wid()
    pltpu.sync_copy(dst_hbm.at[w], idx_v)
    pltpu.sync_copy(ones_hbm, ones_v)

    @pl.when(s == 0)
    def _():
        pltpu.sync_copy(zeros1_hbm, acc_sh)

    plsc.subcore_barrier()

    def body(j, carry):
        pltpu.sync_copy(ones_v, acc_sh.at[idx_v.at[j]], add=True)
        return carry

    lax.fori_loop(0, KC, body, 0)
    plsc.subcore_barrier()

    @pl.when(s == 0)
    def _():
        pltpu.sync_copy(acc_sh, out_hbm.at[c])


@functools.partial(
    pl.kernel,
    out_type=jax.ShapeDtypeStruct((2, N8, H2), jnp.float32),
    mesh=_mesh,
    compiler_params=_sc_params,
    scratch_types=[
        pltpu.VMEM((KC, CHUNK), jnp.int32),
        pltpu.VMEM((KC, CHUNK), jnp.int32),
        pltpu.VMEM((GK, CHUNK, H2), jnp.float32),
        pltpu.VMEM_SHARED((N8, H2), jnp.float32),
        pltpu.SemaphoreType.DMA,
    ],
)
def _sc_agg(y_hbm, src_hbm, dst_hbm, zeros2_hbm, out_hbm,
            src_v, dst_v, rows_v, acc_sh, sem):
    c = lax.axis_index("c")
    s = lax.axis_index("s")
    w = _wid()
    pltpu.sync_copy(src_hbm.at[w], src_v)
    pltpu.sync_copy(dst_hbm.at[w], dst_v)

    @pl.when(s == 0)
    def _():
        pltpu.sync_copy(zeros2_hbm, acc_sh)

    plsc.subcore_barrier()

    def group(g, carry):
        base = g * GK
        descs = []
        for b in range(GK):
            descs.append(
                pltpu.async_copy(y_hbm.at[src_v.at[base + b]], rows_v.at[b], sem))
        for d in descs:
            d.wait()
        for b in range(GK):
            pltpu.sync_copy(rows_v.at[b], acc_sh.at[dst_v.at[base + b]], add=True)
        return carry

    lax.fori_loop(0, KG, group, 0)
    plsc.subcore_barrier()

    @pl.when(s == 0)
    def _():
        pltpu.sync_copy(acc_sh, out_hbm.at[c])


_GROUP = GK * CHUNK


@functools.partial(
    pl.kernel,
    out_type=jax.ShapeDtypeStruct((4, EP, C), jnp.float32),
    mesh=_mesh,
    compiler_params=_sc_params,
    scratch_types=[
        pltpu.VMEM((KC, CHUNK), jnp.int32),
        pltpu.VMEM((_GROUP, C), jnp.float32),
        pltpu.SemaphoreType.DMA,
    ],
)
def _sc_decode_gather(z_hbm, idx_hbm, out_hbm, idx_v, rows_v, sem):
    w = _wid()
    for t in range(4):
        pltpu.sync_copy(idx_hbm.at[t, w], idx_v)

        def group(g, carry, t=t):
            base = g * GK
            descs = []
            for b in range(GK):
                descs.append(
                    pltpu.async_copy(z_hbm.at[idx_v.at[base + b]],
                                     rows_v.at[pl.ds(b * CHUNK, CHUNK)], sem))
            for d in descs:
                d.wait()
            pltpu.sync_copy(rows_v, out_hbm.at[t, pl.ds(w * PT + g * _GROUP, _GROUP)])
            return carry

        lax.fori_loop(0, KG, group, 0)


_BN = 1000
_GN = N // _BN


def _tc_b_body(degt_ref, x_ref, w1_ref, y1_ref):
    deg = degt_ref[:, 0:1] + degt_ref[:, 1:2] + 1.0
    dinv = lax.rsqrt(deg)
    xw = jnp.dot(x_ref[:], w1_ref[:], preferred_element_type=jnp.float32)
    y1_ref[:] = xw * dinv


def _tc_d_body(aggp_ref, degt_ref, y1_ref, b1_ref, y2_ref):
    deg = degt_ref[:, 0:1] + degt_ref[:, 1:2] + 1.0
    dinv = lax.rsqrt(deg)
    agg = aggp_ref[0] + aggp_ref[1] + y1_ref[:]
    h = jnp.maximum(agg * dinv + b1_ref[:], 0.0)
    y2_ref[:] = h * dinv


def _tc_f_body(aggp_ref, degt_ref, y2_ref, wmu_ref, bmu_ref, wls_ref, bls_ref,
               z_ref, kl_ref):
    i = pl.program_id(0)
    deg = degt_ref[:, 0:1] + degt_ref[:, 1:2] + 1.0
    dinv = lax.rsqrt(deg)
    hg = (aggp_ref[0] + aggp_ref[1] + y2_ref[:]) * dinv
    mu = jnp.dot(hg, wmu_ref[:], preferred_element_type=jnp.float32) + bmu_ref[:]
    ls = jnp.minimum(
        jnp.dot(hg, wls_ref[:], preferred_element_type=jnp.float32) + bls_ref[:],
        MAX_LOGSTD)
    z_ref[:] = mu
    klb = jnp.sum(1.0 + 2.0 * ls - mu * mu - jnp.exp(2.0 * ls))

    @pl.when(i == 0)
    def _():
        kl_ref[...] = jnp.zeros((1, 1), jnp.float32)

    kl_ref[...] += klb


_BE = 8000
_GE = E // _BE


def _tc_h_body(r_ref, kl_ref, loss_ref):
    i = pl.program_id(0)
    pos = jnp.sum(r_ref[0] * r_ref[1], axis=1, keepdims=True)
    neg = jnp.sum(r_ref[2] * r_ref[3], axis=1, keepdims=True)
    pls = jnp.sum(jnp.log(jax.nn.sigmoid(pos) + EPS))
    nls = jnp.sum(jnp.log(1.0 - jax.nn.sigmoid(neg) + EPS))

    @pl.when(i == 0)
    def _():
        loss_ref[...] = -0.5 * kl_ref[...] / (N * N)

    loss_ref[...] += -(pls + nls) / E


def _pad_idx(a):
    pad = jnp.full((EP - E,), N, jnp.int32)
    return jnp.concatenate([a.astype(jnp.int32), pad]).reshape(NW, KC, CHUNK)


def kernel(x, edge_index, W1, b1, Wmu, bmu, Wls, bls):
    src = edge_index[0]
    dst = edge_index[1]
    neg = jax.random.randint(jax.random.key(1), (2, E), 0, N, dtype=jnp.int32)
    srcp = _pad_idx(src)
    dstp = _pad_idx(dst)
    dec_idx = jnp.stack([srcp, dstp, _pad_idx(neg[0]), _pad_idx(neg[1])])
    zeros1 = jnp.zeros((N8,), jnp.float32)
    zeros2 = jnp.zeros((N8, H2), jnp.float32)
    ones = jnp.ones((CHUNK,), jnp.float32)

    degp = _sc_degree(dstp, zeros1, ones)
    degt = degp[:, :N].T

    y1 = pl.pallas_call(
        _tc_b_body,
        grid=(_GN,),
        in_specs=[
            pl.BlockSpec((_BN, 2), lambda i: (i, 0)),
            pl.BlockSpec((_BN, D), lambda i: (i, 0)),
            pl.BlockSpec((D, H2), lambda i: (0, 0)),
        ],
        out_specs=pl.BlockSpec((_BN, H2), lambda i: (i, 0)),
        out_shape=jax.ShapeDtypeStruct((N8, H2), jnp.float32),
    )(degt, x, W1)

    agg1 = _sc_agg(y1, srcp, dstp, zeros2)

    y2 = pl.pallas_call(
        _tc_d_body,
        grid=(_GN,),
        in_specs=[
            pl.BlockSpec((2, _BN, H2), lambda i: (0, i, 0)),
            pl.BlockSpec((_BN, 2), lambda i: (i, 0)),
            pl.BlockSpec((_BN, H2), lambda i: (i, 0)),
            pl.BlockSpec((1, H2), lambda i: (0, 0)),
        ],
        out_specs=pl.BlockSpec((_BN, H2), lambda i: (i, 0)),
        out_shape=jax.ShapeDtypeStruct((N8, H2), jnp.float32),
    )(agg1, degt, y1, b1.reshape(1, H2))

    agg2 = _sc_agg(y2, srcp, dstp, zeros2)

    z, kl = pl.pallas_call(
        _tc_f_body,
        grid=(_GN,),
        in_specs=[
            pl.BlockSpec((2, _BN, H2), lambda i: (0, i, 0)),
            pl.BlockSpec((_BN, 2), lambda i: (i, 0)),
            pl.BlockSpec((_BN, H2), lambda i: (i, 0)),
            pl.BlockSpec((H2, C), lambda i: (0, 0)),
            pl.BlockSpec((1, C), lambda i: (0, 0)),
            pl.BlockSpec((H2, C), lambda i: (0, 0)),
            pl.BlockSpec((1, C), lambda i: (0, 0)),
        ],
        out_specs=[
            pl.BlockSpec((_BN, C), lambda i: (i, 0)),
            pl.BlockSpec((1, 1), lambda i: (0, 0)),
        ],
        out_shape=[
            jax.ShapeDtypeStruct((N8, C), jnp.float32),
            jax.ShapeDtypeStruct((1, 1), jnp.float32),
        ],
    )(agg2, degt, y2, Wmu, bmu.reshape(1, C), Wls, bls.reshape(1, C))

    rows = _sc_decode_gather(z, dec_idx)

    loss = pl.pallas_call(
        _tc_h_body,
        grid=(_GE,),
        in_specs=[
            pl.BlockSpec((4, _BE, C), lambda i: (0, i, 0)),
            pl.BlockSpec((1, 1), lambda i: (0, 0)),
        ],
        out_specs=pl.BlockSpec((1, 1), lambda i: (0, 0)),
        out_shape=jax.ShapeDtypeStruct((1, 1), jnp.float32),
    )(rows, kl)

    return (z[:N], loss[0, 0])


# speedup vs baseline: 26.5713x; 2.6536x over previous
import functools

import jax
import jax.numpy as jnp
from jax import lax
from jax.experimental import pallas as pl
from jax.experimental.pallas import tpu as pltpu
from jax.experimental.pallas import tpu_sc as plsc

N = 10000
E = 320000
D = 128
H2 = 32
C = 16
MAX_LOGSTD = 10.0
EPS = 1e-15

N8 = N + 8
NW = 32
CHUNK = 128
KC = 80
PT = KC * CHUNK
EP = NW * PT
GK = 8
KG = KC // GK

_mesh = plsc.VectorSubcoreMesh(core_axis_name="c", subcore_axis_name="s")
_sc_params = pltpu.CompilerParams(use_tc_tiling_on_sc=False,
                                  needs_layout_passes=False)


def _wid():
    return lax.axis_index("s") * 2 + lax.axis_index("c")


@functools.partial(
    pl.kernel,
    out_type=jax.ShapeDtypeStruct((2, N8), jnp.float32),
    mesh=_mesh,
    compiler_params=_sc_params,
    scratch_types=[
        pltpu.VMEM((KC, CHUNK), jnp.int32),
        pltpu.VMEM((CHUNK,), jnp.float32),
        pltpu.VMEM_SHARED((N8,), jnp.float32),
    ],
)
def _sc_degree(dst_hbm, zeros1_hbm, ones_hbm, out_hbm, idx_v, ones_v, acc_sh):
    c = lax.axis_index("c")
    s = lax.axis_index("s")
    w = _wid()
    pltpu.sync_copy(dst_hbm.at[w], idx_v)
    pltpu.sync_copy(ones_hbm, ones_v)

    @pl.when(s == 0)
    def _():
        pltpu.sync_copy(zeros1_hbm, acc_sh)

    plsc.subcore_barrier()

    def body(j, carry):
        pltpu.sync_copy(ones_v, acc_sh.at[idx_v.at[j]], add=True)
        return carry

    lax.fori_loop(0, KC, body, 0)
    plsc.subcore_barrier()

    @pl.when(s == 0)
    def _():
        pltpu.sync_copy(acc_sh, out_hbm.at[c])


@functools.partial(
    pl.kernel,
    out_type=jax.ShapeDtypeStruct((2, N8, H2), jnp.float32),
    mesh=_mesh,
    compiler_params=_sc_params,
    scratch_types=[
        pltpu.VMEM((KC, CHUNK), jnp.int32),
        pltpu.VMEM((KC, CHUNK), jnp.int32),
        pltpu.VMEM((GK, CHUNK, H2), jnp.float32),
        pltpu.VMEM_SHARED((N8, H2), jnp.float32),
        pltpu.SemaphoreType.DMA,
    ],
)
def _sc_agg(y_hbm, src_hbm, dst_hbm, zeros2_hbm, out_hbm,
            src_v, dst_v, rows_v, acc_sh, sem):
    c = lax.axis_index("c")
    s = lax.axis_index("s")
    w = _wid()
    pltpu.sync_copy(src_hbm.at[w], src_v)
    pltpu.sync_copy(dst_hbm.at[w], dst_v)

    @pl.when(s == 0)
    def _():
        pltpu.sync_copy(zeros2_hbm, acc_sh)

    plsc.subcore_barrier()

    def group(g, carry):
        base = g * GK
        descs = []
        for b in range(GK):
            descs.append(
                pltpu.async_copy(y_hbm.at[src_v.at[base + b]], rows_v.at[b], sem))
        for d in descs:
            d.wait()
        for b in range(GK):
            pltpu.sync_copy(rows_v.at[b], acc_sh.at[dst_v.at[base + b]], add=True)
        return carry

    lax.fori_loop(0, KG, group, 0)
    plsc.subcore_barrier()

    @pl.when(s == 0)
    def _():
        pltpu.sync_copy(acc_sh, out_hbm.at[c])


_HF = C // 2


@functools.partial(
    pl.kernel,
    out_type=jax.ShapeDtypeStruct((2, EP), jnp.float32),
    mesh=_mesh,
    compiler_params=_sc_params,
    scratch_types=[
        pltpu.VMEM((_HF * N8,), jnp.float32),
        pltpu.VMEM((PT,), jnp.int32),
        pltpu.VMEM((PT,), jnp.int32),
        pltpu.VMEM((PT,), jnp.float32),
        pltpu.VMEM((PT,), jnp.float32),
    ],
)
def _sc_decode_dots(ztf_hbm, idx_hbm, out_hbm, zt_v, ia_v, ib_v, dp_v, dn_v):
    w = _wid()
    for h in range(2):
        pltpu.sync_copy(ztf_hbm.at[pl.ds(h * _HF * N8, _HF * N8)], zt_v)
        for p, dots_v in ((0, dp_v), (1, dn_v)):
            pltpu.sync_copy(idx_hbm.at[2 * p, w], ia_v)
            pltpu.sync_copy(idx_hbm.at[2 * p + 1, w], ib_v)

            def vec(i, carry, dots_v=dots_v, h=h):
                sl = pl.ds(i * 16, 16)
                ia = ia_v[sl]
                ib = ib_v[sl]
                acc = jnp.zeros((16,), jnp.float32)
                for f in range(_HF):
                    off = jnp.int32(f * N8)
                    fa = plsc.load_gather(zt_v, [ia + off])
                    fb = plsc.load_gather(zt_v, [ib + off])
                    acc = acc + fa * fb
                if h == 0:
                    dots_v[sl] = acc
                else:
                    dots_v[sl] += acc
                return carry

            lax.fori_loop(0, PT // 16, vec, 0)
    pltpu.sync_copy(dp_v, out_hbm.at[0, pl.ds(w * PT, PT)])
    pltpu.sync_copy(dn_v, out_hbm.at[1, pl.ds(w * PT, PT)])


_BN = 1000
_GN = N // _BN


def _tc_b_body(degt_ref, x_ref, w1_ref, y1_ref):
    deg = degt_ref[:, 0:1] + degt_ref[:, 1:2] + 1.0
    dinv = lax.rsqrt(deg)
    xw = jnp.dot(x_ref[:], w1_ref[:], preferred_element_type=jnp.float32)
    y1_ref[:] = xw * dinv


def _tc_d_body(aggp_ref, degt_ref, y1_ref, b1_ref, y2_ref):
    deg = degt_ref[:, 0:1] + degt_ref[:, 1:2] + 1.0
    dinv = lax.rsqrt(deg)
    agg = aggp_ref[0] + aggp_ref[1] + y1_ref[:]
    h = jnp.maximum(agg * dinv + b1_ref[:], 0.0)
    y2_ref[:] = h * dinv


def _tc_f_body(aggp_ref, degt_ref, y2_ref, wmu_ref, bmu_ref, wls_ref, bls_ref,
               z_ref, kl_ref):
    i = pl.program_id(0)
    deg = degt_ref[:, 0:1] + degt_ref[:, 1:2] + 1.0
    dinv = lax.rsqrt(deg)
    hg = (aggp_ref[0] + aggp_ref[1] + y2_ref[:]) * dinv
    mu = jnp.dot(hg, wmu_ref[:], preferred_element_type=jnp.float32) + bmu_ref[:]
    ls = jnp.minimum(
        jnp.dot(hg, wls_ref[:], preferred_element_type=jnp.float32) + bls_ref[:],
        MAX_LOGSTD)
    z_ref[:] = mu
    klb = jnp.sum(1.0 + 2.0 * ls - mu * mu - jnp.exp(2.0 * ls))

    @pl.when(i == 0)
    def _():
        kl_ref[...] = jnp.zeros((1, 1), jnp.float32)

    kl_ref[...] += klb


_BE = 16000
_GE = E // _BE


def _tc_h_body(r_ref, kl_ref, loss_ref):
    i = pl.program_id(0)
    pos = r_ref[0:1, :]
    neg = r_ref[1:2, :]
    pls = jnp.sum(jnp.log(jax.nn.sigmoid(pos) + EPS))
    nls = jnp.sum(jnp.log(1.0 - jax.nn.sigmoid(neg) + EPS))

    @pl.when(i == 0)
    def _():
        loss_ref[...] = -0.5 * kl_ref[...] / (N * N)

    loss_ref[...] += -(pls + nls) / E


def _pad_idx(a):
    pad = jnp.full((EP - E,), N, jnp.int32)
    return jnp.concatenate([a.astype(jnp.int32), pad]).reshape(NW, PT)


def kernel(x, edge_index, W1, b1, Wmu, bmu, Wls, bls):
    src = edge_index[0]
    dst = edge_index[1]
    neg = jax.random.randint(jax.random.key(1), (2, E), 0, N, dtype=jnp.int32)
    srcf = _pad_idx(src)
    dstf = _pad_idx(dst)
    dec_idx = jnp.stack([srcf, dstf, _pad_idx(neg[0]), _pad_idx(neg[1])])
    srcp = srcf.reshape(NW, KC, CHUNK)
    dstp = dstf.reshape(NW, KC, CHUNK)
    zeros1 = jnp.zeros((N8,), jnp.float32)
    zeros2 = jnp.zeros((N8, H2), jnp.float32)
    ones = jnp.ones((CHUNK,), jnp.float32)

    degp = _sc_degree(dstp, zeros1, ones)
    degt = degp[:, :N].T

    y1 = pl.pallas_call(
        _tc_b_body,
        grid=(_GN,),
        in_specs=[
            pl.BlockSpec((_BN, 2), lambda i: (i, 0)),
            pl.BlockSpec((_BN, D), lambda i: (i, 0)),
            pl.BlockSpec((D, H2), lambda i: (0, 0)),
        ],
        out_specs=pl.BlockSpec((_BN, H2), lambda i: (i, 0)),
        out_shape=jax.ShapeDtypeStruct((N8, H2), jnp.float32),
    )(degt, x, W1)

    agg1 = _sc_agg(y1, srcp, dstp, zeros2)

    y2 = pl.pallas_call(
        _tc_d_body,
        grid=(_GN,),
        in_specs=[
            pl.BlockSpec((2, _BN, H2), lambda i: (0, i, 0)),
            pl.BlockSpec((_BN, 2), lambda i: (i, 0)),
            pl.BlockSpec((_BN, H2), lambda i: (i, 0)),
            pl.BlockSpec((1, H2), lambda i: (0, 0)),
        ],
        out_specs=pl.BlockSpec((_BN, H2), lambda i: (i, 0)),
        out_shape=jax.ShapeDtypeStruct((N8, H2), jnp.float32),
    )(agg1, degt, y1, b1.reshape(1, H2))

    agg2 = _sc_agg(y2, srcp, dstp, zeros2)

    z, kl = pl.pallas_call(
        _tc_f_body,
        grid=(_GN,),
        in_specs=[
            pl.BlockSpec((2, _BN, H2), lambda i: (0, i, 0)),
            pl.BlockSpec((_BN, 2), lambda i: (i, 0)),
            pl.BlockSpec((_BN, H2), lambda i: (i, 0)),
            pl.BlockSpec((H2, C), lambda i: (0, 0)),
            pl.BlockSpec((1, C), lambda i: (0, 0)),
            pl.BlockSpec((H2, C), lambda i: (0, 0)),
            pl.BlockSpec((1, C), lambda i: (0, 0)),
        ],
        out_specs=[
            pl.BlockSpec((_BN, C), lambda i: (i, 0)),
            pl.BlockSpec((1, 1), lambda i: (0, 0)),
        ],
        out_shape=[
            jax.ShapeDtypeStruct((N8, C), jnp.float32),
            jax.ShapeDtypeStruct((1, 1), jnp.float32),
        ],
    )(agg2, degt, y2, Wmu, bmu.reshape(1, C), Wls, bls.reshape(1, C))

    ztf = z.T.reshape(-1)
    dots = _sc_decode_dots(ztf, dec_idx)

    loss = pl.pallas_call(
        _tc_h_body,
        grid=(_GE,),
        in_specs=[
            pl.BlockSpec((2, _BE), lambda i: (0, i)),
            pl.BlockSpec((1, 1), lambda i: (0, 0)),
        ],
        out_specs=pl.BlockSpec((1, 1), lambda i: (0, 0)),
        out_shape=jax.ShapeDtypeStruct((1, 1), jnp.float32),
    )(dots, kl)

    return (z[:N], loss[0, 0])


# speedup vs baseline: 33.4112x; 1.2574x over previous
import functools

import jax
import jax.numpy as jnp
from jax import lax
from jax.experimental import pallas as pl
from jax.experimental.pallas import tpu as pltpu
from jax.experimental.pallas import tpu_sc as plsc

N = 10000
E = 320000
D = 128
H2 = 32
C = 16
MAX_LOGSTD = 10.0
EPS = 1e-15

N8 = N + 8
NW = 32
CHUNK = 128
KC = 80
PT = KC * CHUNK
EP = NW * PT
GK = 8
KG = KC // GK

_mesh = plsc.VectorSubcoreMesh(core_axis_name="c", subcore_axis_name="s")
_sc_params = pltpu.CompilerParams(use_tc_tiling_on_sc=False,
                                  needs_layout_passes=False)


def _wid():
    return lax.axis_index("s") * 2 + lax.axis_index("c")


@functools.partial(
    pl.kernel,
    out_type=jax.ShapeDtypeStruct((2, N8), jnp.float32),
    mesh=_mesh,
    compiler_params=_sc_params,
    scratch_types=[
        pltpu.VMEM((KC, CHUNK), jnp.int32),
        pltpu.VMEM((CHUNK,), jnp.float32),
        pltpu.VMEM_SHARED((N8,), jnp.float32),
    ],
)
def _sc_degree(dst_hbm, zeros1_hbm, ones_hbm, out_hbm, idx_v, ones_v, acc_sh):
    c = lax.axis_index("c")
    s = lax.axis_index("s")
    w = _wid()
    pltpu.sync_copy(dst_hbm.at[w], idx_v)
    pltpu.sync_copy(ones_hbm, ones_v)

    @pl.when(s == 0)
    def _():
        pltpu.sync_copy(zeros1_hbm, acc_sh)

    plsc.subcore_barrier()

    def body(j, carry):
        pltpu.sync_copy(ones_v, acc_sh.at[idx_v.at[j]], add=True)
        return carry

    lax.fori_loop(0, KC, body, 0)
    plsc.subcore_barrier()

    @pl.when(s == 0)
    def _():
        pltpu.sync_copy(acc_sh, out_hbm.at[c])


@functools.partial(
    pl.kernel,
    out_type=jax.ShapeDtypeStruct((2, N8, H2), jnp.float32),
    mesh=_mesh,
    compiler_params=_sc_params,
    scratch_types=[
        pltpu.VMEM((KC, CHUNK), jnp.int32),
        pltpu.VMEM((KC, CHUNK), jnp.int32),
        pltpu.VMEM((GK, CHUNK, H2), jnp.float32),
        pltpu.VMEM_SHARED((N8, H2), jnp.float32),
        pltpu.VMEM_SHARED((N8, H2), jnp.float32),
        pltpu.SemaphoreType.DMA,
    ],
)
def _sc_agg(y_hbm, src_hbm, dst_hbm, zeros2_hbm, out_hbm,
            src_v, dst_v, rows_v, y_sh, acc_sh, sem):
    c = lax.axis_index("c")
    s = lax.axis_index("s")
    w = _wid()
    pltpu.sync_copy(src_hbm.at[w], src_v)
    pltpu.sync_copy(dst_hbm.at[w], dst_v)

    @pl.when(s == 0)
    def _():
        pltpu.sync_copy(y_hbm, y_sh)

    @pl.when(s == 1)
    def _():
        pltpu.sync_copy(zeros2_hbm, acc_sh)

    plsc.subcore_barrier()

    def group(g, carry):
        base = g * GK
        descs = []
        for b in range(GK):
            descs.append(
                pltpu.async_copy(y_sh.at[src_v.at[base + b]], rows_v.at[b], sem))
        for d in descs:
            d.wait()
        for b in range(GK):
            pltpu.sync_copy(rows_v.at[b], acc_sh.at[dst_v.at[base + b]], add=True)
        return carry

    lax.fori_loop(0, KG, group, 0)
    plsc.subcore_barrier()

    @pl.when(s == 0)
    def _():
        pltpu.sync_copy(acc_sh, out_hbm.at[c])


_HF = C // 2


@functools.partial(
    pl.kernel,
    out_type=jax.ShapeDtypeStruct((2, EP), jnp.float32),
    mesh=_mesh,
    compiler_params=_sc_params,
    scratch_types=[
        pltpu.VMEM((_HF * N8,), jnp.float32),
        pltpu.VMEM((PT,), jnp.int32),
        pltpu.VMEM((PT,), jnp.int32),
        pltpu.VMEM((PT,), jnp.float32),
        pltpu.VMEM((PT,), jnp.float32),
    ],
)
def _sc_decode_dots(ztf_hbm, idx_hbm, out_hbm, zt_v, ia_v, ib_v, dp_v, dn_v):
    w = _wid()
    for h in range(2):
        pltpu.sync_copy(ztf_hbm.at[pl.ds(h * _HF * N8, _HF * N8)], zt_v)
        for p, dots_v in ((0, dp_v), (1, dn_v)):
            pltpu.sync_copy(idx_hbm.at[2 * p, w], ia_v)
            pltpu.sync_copy(idx_hbm.at[2 * p + 1, w], ib_v)

            def vec(i, carry, dots_v=dots_v, h=h):
                sl = pl.ds(i * 16, 16)
                ia = ia_v[sl]
                ib = ib_v[sl]
                acc = jnp.zeros((16,), jnp.float32)
                for f in range(_HF):
                    off = jnp.int32(f * N8)
                    fa = plsc.load_gather(zt_v, [ia + off])
                    fb = plsc.load_gather(zt_v, [ib + off])
                    acc = acc + fa * fb
                if h == 0:
                    dots_v[sl] = acc
                else:
                    dots_v[sl] += acc
                return carry

            lax.fori_loop(0, PT // 16, vec, 0)
    pltpu.sync_copy(dp_v, out_hbm.at[0, pl.ds(w * PT, PT)])
    pltpu.sync_copy(dn_v, out_hbm.at[1, pl.ds(w * PT, PT)])


_BN = 1000
_GN = N // _BN


def _tc_b_body(degt_ref, x_ref, w1_ref, y1_ref):
    deg = degt_ref[:, 0:1] + degt_ref[:, 1:2] + 1.0
    dinv = lax.rsqrt(deg)
    xw = jnp.dot(x_ref[:], w1_ref[:], preferred_element_type=jnp.float32)
    y1_ref[:] = xw * dinv


def _tc_d_body(aggp_ref, degt_ref, y1_ref, b1_ref, y2_ref):
    deg = degt_ref[:, 0:1] + degt_ref[:, 1:2] + 1.0
    dinv = lax.rsqrt(deg)
    agg = aggp_ref[0] + aggp_ref[1] + y1_ref[:]
    h = jnp.maximum(agg * dinv + b1_ref[:], 0.0)
    y2_ref[:] = h * dinv


def _tc_f_body(aggp_ref, degt_ref, y2_ref, wmu_ref, bmu_ref, wls_ref, bls_ref,
               z_ref, kl_ref):
    i = pl.program_id(0)
    deg = degt_ref[:, 0:1] + degt_ref[:, 1:2] + 1.0
    dinv = lax.rsqrt(deg)
    hg = (aggp_ref[0] + aggp_ref[1] + y2_ref[:]) * dinv
    mu = jnp.dot(hg, wmu_ref[:], preferred_element_type=jnp.float32) + bmu_ref[:]
    ls = jnp.minimum(
        jnp.dot(hg, wls_ref[:], preferred_element_type=jnp.float32) + bls_ref[:],
        MAX_LOGSTD)
    z_ref[:] = mu
    klb = jnp.sum(1.0 + 2.0 * ls - mu * mu - jnp.exp(2.0 * ls))

    @pl.when(i == 0)
    def _():
        kl_ref[...] = jnp.zeros((1, 1), jnp.float32)

    kl_ref[...] += klb


_BE = 16000
_GE = E // _BE


def _tc_h_body(r_ref, kl_ref, loss_ref):
    i = pl.program_id(0)
    pos = r_ref[0:1, :]
    neg = r_ref[1:2, :]
    pls = jnp.sum(jnp.log(jax.nn.sigmoid(pos) + EPS))
    nls = jnp.sum(jnp.log(1.0 - jax.nn.sigmoid(neg) + EPS))

    @pl.when(i == 0)
    def _():
        loss_ref[...] = -0.5 * kl_ref[...] / (N * N)

    loss_ref[...] += -(pls + nls) / E


def _pad_idx(a):
    pad = jnp.full((EP - E,), N, jnp.int32)
    return jnp.concatenate([a.astype(jnp.int32), pad]).reshape(NW, PT)


def kernel(x, edge_index, W1, b1, Wmu, bmu, Wls, bls):
    src = edge_index[0]
    dst = edge_index[1]
    neg = jax.random.randint(jax.random.key(1), (2, E), 0, N, dtype=jnp.int32)
    srcf = _pad_idx(src)
    dstf = _pad_idx(dst)
    dec_idx = jnp.stack([srcf, dstf, _pad_idx(neg[0]), _pad_idx(neg[1])])
    srcp = srcf.reshape(NW, KC, CHUNK)
    dstp = dstf.reshape(NW, KC, CHUNK)
    zeros1 = jnp.zeros((N8,), jnp.float32)
    zeros2 = jnp.zeros((N8, H2), jnp.float32)
    ones = jnp.ones((CHUNK,), jnp.float32)

    degp = _sc_degree(dstp, zeros1, ones)
    degt = degp[:, :N].T

    y1 = pl.pallas_call(
        _tc_b_body,
        grid=(_GN,),
        in_specs=[
            pl.BlockSpec((_BN, 2), lambda i: (i, 0)),
            pl.BlockSpec((_BN, D), lambda i: (i, 0)),
            pl.BlockSpec((D, H2), lambda i: (0, 0)),
        ],
        out_specs=pl.BlockSpec((_BN, H2), lambda i: (i, 0)),
        out_shape=jax.ShapeDtypeStruct((N8, H2), jnp.float32),
    )(degt, x, W1)

    agg1 = _sc_agg(y1, srcp, dstp, zeros2)

    y2 = pl.pallas_call(
        _tc_d_body,
        grid=(_GN,),
        in_specs=[
            pl.BlockSpec((2, _BN, H2), lambda i: (0, i, 0)),
            pl.BlockSpec((_BN, 2), lambda i: (i, 0)),
            pl.BlockSpec((_BN, H2), lambda i: (i, 0)),
            pl.BlockSpec((1, H2), lambda i: (0, 0)),
        ],
        out_specs=pl.BlockSpec((_BN, H2), lambda i: (i, 0)),
        out_shape=jax.ShapeDtypeStruct((N8, H2), jnp.float32),
    )(agg1, degt, y1, b1.reshape(1, H2))

    agg2 = _sc_agg(y2, srcp, dstp, zeros2)

    z, kl = pl.pallas_call(
        _tc_f_body,
        grid=(_GN,),
        in_specs=[
            pl.BlockSpec((2, _BN, H2), lambda i: (0, i, 0)),
            pl.BlockSpec((_BN, 2), lambda i: (i, 0)),
            pl.BlockSpec((_BN, H2), lambda i: (i, 0)),
            pl.BlockSpec((H2, C), lambda i: (0, 0)),
            pl.BlockSpec((1, C), lambda i: (0, 0)),
            pl.BlockSpec((H2, C), lambda i: (0, 0)),
            pl.BlockSpec((1, C), lambda i: (0, 0)),
        ],
        out_specs=[
            pl.BlockSpec((_BN, C), lambda i: (i, 0)),
            pl.BlockSpec((1, 1), lambda i: (0, 0)),
        ],
        out_shape=[
            jax.ShapeDtypeStruct((N8, C), jnp.float32),
            jax.ShapeDtypeStruct((1, 1), jnp.float32),
        ],
    )(agg2, degt, y2, Wmu, bmu.reshape(1, C), Wls, bls.reshape(1, C))

    ztf = z.T.reshape(-1)
    dots = _sc_decode_dots(ztf, dec_idx)

    loss = pl.pallas_call(
        _tc_h_body,
        grid=(_GE,),
        in_specs=[
            pl.BlockSpec((2, _BE), lambda i: (0, i)),
            pl.BlockSpec((1, 1), lambda i: (0, 0)),
        ],
        out_specs=pl.BlockSpec((1, 1), lambda i: (0, 0)),
        out_shape=jax.ShapeDtypeStruct((1, 1), jnp.float32),
    )(dots, kl)

    return (z[:N], loss[0, 0])
